# Initial kernel scaffold; baseline (speedup 1.0000x reference)
import jax
import jax.numpy as jnp
from jax.experimental import pallas as pl


def kernel(downsampling_features, swinT_features, Wt, bt, Wct0, bct0, Wc0, bc0, Wct1, bct1, Wc1, bc1, Wg, bg, in_proj_w, in_proj_b, out_w, out_b):
    raise NotImplementedError("write your pallas kernel here")


# speedup vs baseline: 2.0419x; 2.0419x over previous
import functools
import math

import jax
import jax.numpy as jnp
from jax import lax
from jax.experimental import pallas as pl
from jax.experimental.pallas import tpu as pltpu
from jax.experimental.pallas import tpu_sc as plsc

NUM_HEADS = 8
K_TOP = 1000
K_PAD = 1024
C = 96
CP = 128
HW = 50176

_SC_NC = 2
_SC_NS = 16
_SC_NW = _SC_NC * _SC_NS


def _gate_body(x_ref, w_ref, o_ref):
  o_ref[...] = jnp.dot(w_ref[...], x_ref[...],
                       preferred_element_type=jnp.float32)


def _gate_scores(down_cm, wg8):
  grid = 8
  blk = HW // grid
  return pl.pallas_call(
      _gate_body,
      grid=(grid,),
      in_specs=[
          pl.BlockSpec((C, blk), lambda j: (0, j)),
          pl.BlockSpec((8, C), lambda j: (0, 0)),
      ],
      out_specs=pl.BlockSpec((8, blk), lambda j: (0, j)),
      out_shape=jax.ShapeDtypeStruct((8, HW), jnp.float32),
  )(down_cm, wg8)


def _mm_bias_body(x_ref, w_ref, b_ref, o_ref):
  o_ref[...] = jnp.dot(x_ref[...], w_ref[...],
                       preferred_element_type=jnp.float32) + b_ref[...]


def _conv1x1(x, w, b):
  n, cin = x.shape
  cout = w.shape[1]
  return pl.pallas_call(
      _mm_bias_body,
      out_shape=jax.ShapeDtypeStruct((n, cout), jnp.float32),
  )(x, w, b)


def _convt_small_body(x_ref, w_ref, b_ref, o_ref):
  for p in range(4):
    py, px = p // 2, p % 2
    acc = jnp.zeros((56 * 56, C), jnp.float32)
    for a in range(2):
      for bb in range(2):
        ry, rx = a - 1 + py, bb - 1 + px
        ky, kx = 2 * a + py, 2 * bb + px
        xs = x_ref[pl.ds(1 + ry, 56), pl.ds(1 + rx, 56), :]
        acc = acc + jnp.dot(xs.reshape(56 * 56, C), w_ref[ky, kx],
                            preferred_element_type=jnp.float32)
    o_ref[p] = jnp.maximum(acc + b_ref[...], 0.0).reshape(56, 56, C)


def _convt0(xp, w4, b):
  return pl.pallas_call(
      _convt_small_body,
      out_shape=jax.ShapeDtypeStruct((4, 56, 56, C), jnp.float32),
  )(xp, w4, b)


def _conv3x3_body(x_ref, w_ref, b_ref, o_ref):
  acc = jnp.zeros((112 * 112, C), jnp.float32)
  for dy in range(3):
    for dx in range(3):
      xs = x_ref[pl.ds(dy, 112), pl.ds(dx, 112), :]
      acc = acc + jnp.dot(xs.reshape(112 * 112, C), w_ref[dy, dx],
                          preferred_element_type=jnp.float32)
  o_ref[...] = (acc + b_ref[...]).reshape(112, 112, C)


def _conv3x3_mid(xp, w9, b):
  return pl.pallas_call(
      _conv3x3_body,
      out_shape=jax.ShapeDtypeStruct((112, 112, C), jnp.float32),
  )(xp, w9, b)


def _convt_big_body(x_ref, w_ref, b_ref, o_ref):
  p = pl.program_id(0)
  py, px = p // 2, p % 2
  acc = jnp.zeros((112 * 112, C), jnp.float32)
  for a in range(2):
    for bb in range(2):
      xs = x_ref[pl.ds(a + py, 112), pl.ds(bb + px, 112), :]
      acc = acc + jnp.dot(xs.reshape(112 * 112, C), w_ref[0, a, bb],
                          preferred_element_type=jnp.float32)
  o_ref[0] = jnp.maximum(acc + b_ref[...], 0.0).reshape(112, 112, C)


def _convt1(xp, wp, b):
  return pl.pallas_call(
      _convt_big_body,
      grid=(4,),
      in_specs=[
          pl.BlockSpec((114, 114, C), lambda p: (0, 0, 0)),
          pl.BlockSpec((1, 2, 2, C, C), lambda p: (p, 0, 0, 0, 0)),
          pl.BlockSpec((1, C), lambda p: (0, 0)),
      ],
      out_specs=pl.BlockSpec((1, 112, 112, C), lambda p: (p, 0, 0, 0)),
      out_shape=jax.ShapeDtypeStruct((4, 112, 112, C), jnp.float32),
  )(xp, wp, b)


def _patch_mm_body(x_ref, m_ref, w_ref, b_ref, o_ref):
  o_ref[...] = jnp.dot(x_ref[...] * m_ref[...], w_ref[...],
                       preferred_element_type=jnp.float32) + b_ref[...]


def _patch_mm(patches, mask, w, b):
  return pl.pallas_call(
      _patch_mm_body,
      out_shape=jax.ShapeDtypeStruct((K_PAD, C), jnp.float32),
  )(patches, mask, w, b)


def _mha_body(d_ref, s_ref, wq_ref, wkv_ref, bq_ref, bkv_ref, wo_ref,
              bo_ref, o_ref):
  hd = C // NUM_HEADS
  scale = 1.0 / math.sqrt(hd)
  q = jnp.dot(d_ref[...], wq_ref[...],
              preferred_element_type=jnp.float32) + bq_ref[...]
  kv = jnp.dot(s_ref[...], wkv_ref[...],
               preferred_element_type=jnp.float32) + bkv_ref[...]
  col = lax.broadcasted_iota(jnp.int32, (K_PAD, K_PAD), 1)
  kv_mask = col < K_TOP
  outs = []
  for h in range(NUM_HEADS):
    qh = q[:, h * hd:(h + 1) * hd]
    kh = kv[:, h * hd:(h + 1) * hd]
    vh = kv[:, C + h * hd:C + (h + 1) * hd]
    s = lax.dot_general(qh, kh, (((1,), (1,)), ((), ())),
                        preferred_element_type=jnp.float32) * scale
    s = jnp.where(kv_mask, s, -1e30)
    m = jnp.max(s, axis=-1, keepdims=True)
    e = jnp.exp(s - m)
    p = e / jnp.sum(e, axis=-1, keepdims=True)
    outs.append(jnp.dot(p, vh, preferred_element_type=jnp.float32))
  o = jnp.concatenate(outs, axis=1)
  o_ref[...] = jnp.dot(o, wo_ref[...],
                       preferred_element_type=jnp.float32) + bo_ref[...]


def _mha(d_sub, s_sub, wq, wkv, bq, bkv, wo, bo):
  return pl.pallas_call(
      _mha_body,
      out_shape=jax.ShapeDtypeStruct((K_PAD, C), jnp.float32),
  )(d_sub, s_sub, wq, wkv, bq, bkv, wo, bo)


def _sc_mesh():
  return plsc.VectorSubcoreMesh(core_axis_name="c", subcore_axis_name="s",
                                num_cores=_SC_NC, num_subcores=_SC_NS)


@functools.partial(jax.jit, static_argnums=(2,))
def _sc_gather(table, idx, chunk):
  b = idx.shape[0]
  per_w = b // _SC_NW
  chunks = per_w // chunk

  @functools.partial(
      pl.kernel,
      mesh=_sc_mesh(),
      out_type=jax.ShapeDtypeStruct((b, CP), jnp.float32),
      scratch_types=[
          pltpu.VMEM((chunk,), jnp.int32),
          pltpu.VMEM((chunk, CP), jnp.float32),
          pltpu.SemaphoreType.DMA,
      ],
  )
  def gk(table_hbm, idx_hbm, out_hbm, idx_v, rows_v, sem):
    wid = lax.axis_index("s") * _SC_NC + lax.axis_index("c")
    base = wid * per_w
    for ci in range(chunks):
      off = base + ci * chunk
      pltpu.sync_copy(idx_hbm.at[pl.ds(off, chunk)], idx_v)
      pltpu.async_copy(table_hbm.at[idx_v], rows_v, sem).wait()
      pltpu.sync_copy(rows_v, out_hbm.at[pl.ds(off, chunk)])

  return gk(table, idx)


def _sc_scatter(dest_ref, att, idx):
  per_w = K_PAD // _SC_NW

  @functools.partial(
      pl.kernel,
      mesh=_sc_mesh(),
      out_type=(),
      scratch_types=[
          pltpu.VMEM((per_w,), jnp.int32),
          pltpu.VMEM((per_w, CP), jnp.float32),
          pltpu.SemaphoreType.DMA,
      ],
  )
  def sk(att_hbm, idx_hbm, dest_hbm, idx_v, rows_v, sem):
    wid = lax.axis_index("s") * _SC_NC + lax.axis_index("c")
    base = wid * per_w
    pltpu.sync_copy(idx_hbm.at[pl.ds(base, per_w)], idx_v)
    pltpu.sync_copy(att_hbm.at[pl.ds(base, per_w)], rows_v)
    pltpu.async_copy(rows_v, dest_hbm.at[idx_v], sem).wait()

  sk(att, idx, dest_ref)


def kernel(downsampling_features, swinT_features, Wt, bt, Wct0, bct0, Wc0,
           bc0, Wct1, bct1, Wc1, bc1, Wg, bg, in_proj_w, in_proj_b, out_w,
           out_b):
  down_cm = downsampling_features.reshape(C, HW)

  wg8 = jnp.zeros((8, C), jnp.float32).at[0].set(Wg[0])
  scores = _gate_scores(down_cm, wg8)
  top_idx = lax.top_k(scores[0], K_TOP)[1]
  idx_p = jnp.concatenate(
      [top_idx, jnp.broadcast_to(top_idx[K_TOP - 1:K_TOP], (K_PAD - K_TOP,))])

  down_hwc = jnp.pad(down_cm.T, ((0, 0), (0, CP - C)))
  d_sub = _sc_gather(down_hwc, idx_p, K_PAD // _SC_NW)

  sw_in = swinT_features.reshape(384, 56 * 56).T
  t0 = _conv1x1(sw_in, Wt.T, bt.reshape(1, C)).reshape(56, 56, C)
  t0p = jnp.pad(t0, ((1, 1), (1, 1), (0, 0)))
  w4_0 = Wct0.transpose(2, 3, 1, 0)
  u1ph = _convt0(t0p, w4_0, bct0.reshape(1, C))
  u1 = u1ph.reshape(2, 2, 56, 56, C).transpose(2, 0, 3, 1, 4).reshape(
      112, 112, C)
  u1p = jnp.pad(u1, ((1, 1), (1, 1), (0, 0)))
  t1 = _conv3x3_mid(u1p, Wc0.transpose(2, 3, 1, 0), bc0.reshape(1, C))
  t1p = jnp.pad(t1, ((1, 1), (1, 1), (0, 0)))

  w4_1 = Wct1.transpose(2, 3, 1, 0)
  wp = jnp.stack([
      w4_1[py::2][:2][:, px::2][:, :2]
      for py in range(2) for px in range(2)
  ])
  u3ph = _convt1(t1p, wp, bct1.reshape(1, C))
  u3flat = jnp.pad(u3ph.reshape(HW, C), ((0, 0), (0, CP - C)))

  y = idx_p // 224
  x = idx_p % 224
  d3 = jnp.arange(3, dtype=jnp.int32)
  yy = y[:, None, None] + d3[None, :, None] - 1
  xx = x[:, None, None] + d3[None, None, :] - 1
  valid = (yy >= 0) & (yy < 224) & (xx >= 0) & (xx < 224)
  yyc = jnp.clip(yy, 0, 223)
  xxc = jnp.clip(xx, 0, 223)
  ph = (yyc % 2) * 2 + (xxc % 2)
  gidx = (ph * (112 * 112) + (yyc // 2) * 112 + (xxc // 2)).reshape(-1)
  patches = _sc_gather(u3flat, gidx, 96).reshape(K_PAD, 9 * CP)
  mask = jnp.repeat(valid.reshape(K_PAD, 9).astype(jnp.float32), CP, axis=1)
  w3f = jnp.pad(Wc1.transpose(2, 3, 1, 0),
                ((0, 0), (0, 0), (0, CP - C), (0, 0))).reshape(9 * CP, C)
  s_sub = _patch_mm(patches, mask, w3f, bc1.reshape(1, C))

  wq = jnp.pad(in_proj_w[:C].T, ((0, CP - C), (0, 0)))
  wkv = in_proj_w[C:].T
  bq = in_proj_b[:C].reshape(1, C)
  bkv = in_proj_b[C:].reshape(1, 2 * C)
  att = _mha(d_sub, s_sub, wq, wkv, bq, bkv, out_w.T, out_b.reshape(1, C))

  att_p = jnp.pad(att, ((0, 0), (0, CP - C)))
  dest = jax.new_ref(down_hwc)
  _sc_scatter(dest, att_p, idx_p)
  attended = dest[...]
  return attended[:, :C].T.reshape(1, C, 224, 224)


# speedup vs baseline: 2.1271x; 1.0417x over previous
import functools
import math

import jax
import jax.numpy as jnp
from jax import lax
from jax.experimental import pallas as pl
from jax.experimental.pallas import tpu as pltpu
from jax.experimental.pallas import tpu_sc as plsc

NUM_HEADS = 8
K_TOP = 1000
K_PAD = 1024
C = 96
CP = 128
HW = 50176

_SC_NC = 2
_SC_NS = 16
_SC_NW = _SC_NC * _SC_NS


def _gate_body(x_ref, w_ref, s_ref, t_ref):
  x = x_ref[...]
  s_ref[...] = jnp.dot(w_ref[...], x, preferred_element_type=jnp.float32)
  t_ref[...] = jnp.concatenate(
      [x.T, jnp.zeros((x.shape[1], CP - C), jnp.float32)], axis=1)


def _gate_scores(down_cm, wg8):
  grid = 14
  blk = HW // grid
  return pl.pallas_call(
      _gate_body,
      grid=(grid,),
      in_specs=[
          pl.BlockSpec((C, blk), lambda j: (0, j)),
          pl.BlockSpec((8, C), lambda j: (0, 0)),
      ],
      out_specs=[
          pl.BlockSpec((8, blk), lambda j: (0, j)),
          pl.BlockSpec((blk, CP), lambda j: (j, 0)),
      ],
      out_shape=[
          jax.ShapeDtypeStruct((8, HW), jnp.float32),
          jax.ShapeDtypeStruct((HW, CP), jnp.float32),
      ],
  )(down_cm, wg8)


def _untranspose_body(x_ref, o_ref):
  o_ref[...] = x_ref[:, :C].T


def _untranspose(x):
  grid = 14
  blk = HW // grid
  return pl.pallas_call(
      _untranspose_body,
      grid=(grid,),
      in_specs=[pl.BlockSpec((blk, CP), lambda j: (j, 0))],
      out_specs=pl.BlockSpec((C, blk), lambda j: (0, j)),
      out_shape=jax.ShapeDtypeStruct((C, HW), jnp.float32),
  )(x)


def _mm_bias_body(x_ref, w_ref, b_ref, o_ref):
  o_ref[...] = jnp.dot(x_ref[...], w_ref[...],
                       preferred_element_type=jnp.float32) + b_ref[...]


def _conv1x1(x, w, b):
  n, cin = x.shape
  cout = w.shape[1]
  return pl.pallas_call(
      _mm_bias_body,
      out_shape=jax.ShapeDtypeStruct((n, cout), jnp.float32),
  )(x, w, b)


def _convt_small_body(x_ref, w_ref, b_ref, o_ref):
  for p in range(4):
    py, px = p // 2, p % 2
    acc = jnp.zeros((56 * 56, C), jnp.float32)
    for a in range(2):
      for bb in range(2):
        ry, rx = a - 1 + py, bb - 1 + px
        ky, kx = 2 * a + py, 2 * bb + px
        xs = x_ref[pl.ds(1 + ry, 56), pl.ds(1 + rx, 56), :]
        acc = acc + jnp.dot(xs.reshape(56 * 56, C), w_ref[ky, kx],
                            preferred_element_type=jnp.float32)
    o_ref[p] = jnp.maximum(acc + b_ref[...], 0.0).reshape(56, 56, C)


def _convt0(xp, w4, b):
  return pl.pallas_call(
      _convt_small_body,
      out_shape=jax.ShapeDtypeStruct((4, 56, 56, C), jnp.float32),
  )(xp, w4, b)


def _conv3x3_body(x_ref, w_ref, b_ref, o_ref):
  acc = jnp.zeros((112 * 112, C), jnp.float32)
  for dy in range(3):
    for dx in range(3):
      xs = x_ref[pl.ds(dy, 112), pl.ds(dx, 112), :]
      acc = acc + jnp.dot(xs.reshape(112 * 112, C), w_ref[dy, dx],
                          preferred_element_type=jnp.float32)
  o_ref[...] = (acc + b_ref[...]).reshape(112, 112, C)


def _conv3x3_mid(xp, w9, b):
  return pl.pallas_call(
      _conv3x3_body,
      out_shape=jax.ShapeDtypeStruct((112, 112, C), jnp.float32),
  )(xp, w9, b)


def _convt_big_body(x_ref, w_ref, b_ref, o_ref):
  p = pl.program_id(0)
  py, px = p // 2, p % 2
  acc = jnp.zeros((112 * 112, CP), jnp.float32)
  for a in range(2):
    for bb in range(2):
      xs = x_ref[pl.ds(a + py, 112), pl.ds(bb + px, 112), :]
      acc = acc + jnp.dot(xs.reshape(112 * 112, C), w_ref[0, a, bb],
                          preferred_element_type=jnp.float32)
  o_ref[0] = jnp.maximum(acc + b_ref[...], 0.0).reshape(112, 112, CP)


def _convt1(xp, wp, b):
  return pl.pallas_call(
      _convt_big_body,
      grid=(4,),
      in_specs=[
          pl.BlockSpec((114, 114, C), lambda p: (0, 0, 0)),
          pl.BlockSpec((1, 2, 2, C, CP), lambda p: (p, 0, 0, 0, 0)),
          pl.BlockSpec((1, CP), lambda p: (0, 0)),
      ],
      out_specs=pl.BlockSpec((1, 112, 112, CP), lambda p: (p, 0, 0, 0)),
      out_shape=jax.ShapeDtypeStruct((4, 112, 112, CP), jnp.float32),
  )(xp, wp, b)


def _patch_mm_body(x_ref, m_ref, w_ref, b_ref, o_ref):
  o_ref[...] = jnp.dot(x_ref[...] * m_ref[...], w_ref[...],
                       preferred_element_type=jnp.float32) + b_ref[...]


def _patch_mm(patches, mask, w, b):
  return pl.pallas_call(
      _patch_mm_body,
      out_shape=jax.ShapeDtypeStruct((K_PAD, C), jnp.float32),
  )(patches, mask, w, b)


def _mha_body(d_ref, s_ref, wq_ref, wkv_ref, bq_ref, bkv_ref, wo_ref,
              bo_ref, o_ref):
  hd = C // NUM_HEADS
  scale = 1.0 / math.sqrt(hd)
  q = jnp.dot(d_ref[...], wq_ref[...],
              preferred_element_type=jnp.float32) + bq_ref[...]
  kv = jnp.dot(s_ref[...], wkv_ref[...],
               preferred_element_type=jnp.float32) + bkv_ref[...]
  col = lax.broadcasted_iota(jnp.int32, (K_PAD, K_PAD), 1)
  kv_mask = col < K_TOP
  outs = []
  for h in range(NUM_HEADS):
    qh = q[:, h * hd:(h + 1) * hd]
    kh = kv[:, h * hd:(h + 1) * hd]
    vh = kv[:, C + h * hd:C + (h + 1) * hd]
    s = lax.dot_general(qh, kh, (((1,), (1,)), ((), ())),
                        preferred_element_type=jnp.float32) * scale
    s = jnp.where(kv_mask, s, -1e30)
    m = jnp.max(s, axis=-1, keepdims=True)
    e = jnp.exp(s - m)
    p = e / jnp.sum(e, axis=-1, keepdims=True)
    outs.append(jnp.dot(p, vh, preferred_element_type=jnp.float32))
  o = jnp.concatenate(outs, axis=1)
  o_ref[...] = jnp.dot(o, wo_ref[...],
                       preferred_element_type=jnp.float32) + bo_ref[...]


def _mha(d_sub, s_sub, wq, wkv, bq, bkv, wo, bo):
  return pl.pallas_call(
      _mha_body,
      out_shape=jax.ShapeDtypeStruct((K_PAD, C), jnp.float32),
  )(d_sub, s_sub, wq, wkv, bq, bkv, wo, bo)


def _sc_mesh():
  return plsc.VectorSubcoreMesh(core_axis_name="c", subcore_axis_name="s",
                                num_cores=_SC_NC, num_subcores=_SC_NS)


@functools.partial(jax.jit, static_argnums=(2,))
def _sc_gather(table, idx, chunk):
  b = idx.shape[0]
  per_w = b // _SC_NW
  chunks = per_w // chunk

  @functools.partial(
      pl.kernel,
      mesh=_sc_mesh(),
      out_type=jax.ShapeDtypeStruct((b, CP), jnp.float32),
      scratch_types=[
          pltpu.VMEM((chunk,), jnp.int32),
          pltpu.VMEM((chunk, CP), jnp.float32),
          pltpu.SemaphoreType.DMA,
      ],
  )
  def gk(table_hbm, idx_hbm, out_hbm, idx_v, rows_v, sem):
    wid = lax.axis_index("s") * _SC_NC + lax.axis_index("c")
    base = wid * per_w
    for ci in range(chunks):
      off = base + ci * chunk
      pltpu.sync_copy(idx_hbm.at[pl.ds(off, chunk)], idx_v)
      pltpu.async_copy(table_hbm.at[idx_v], rows_v, sem).wait()
      pltpu.sync_copy(rows_v, out_hbm.at[pl.ds(off, chunk)])

  return gk(table, idx)


def _sc_scatter(dest_ref, att, idx):
  per_w = K_PAD // _SC_NW

  @functools.partial(
      pl.kernel,
      mesh=_sc_mesh(),
      out_type=(),
      scratch_types=[
          pltpu.VMEM((per_w,), jnp.int32),
          pltpu.VMEM((per_w, CP), jnp.float32),
          pltpu.SemaphoreType.DMA,
      ],
  )
  def sk(att_hbm, idx_hbm, dest_hbm, idx_v, rows_v, sem):
    wid = lax.axis_index("s") * _SC_NC + lax.axis_index("c")
    base = wid * per_w
    pltpu.sync_copy(idx_hbm.at[pl.ds(base, per_w)], idx_v)
    pltpu.sync_copy(att_hbm.at[pl.ds(base, per_w)], rows_v)
    pltpu.async_copy(rows_v, dest_hbm.at[idx_v], sem).wait()

  sk(att, idx, dest_ref)


def kernel(downsampling_features, swinT_features, Wt, bt, Wct0, bct0, Wc0,
           bc0, Wct1, bct1, Wc1, bc1, Wg, bg, in_proj_w, in_proj_b, out_w,
           out_b):
  down_cm = downsampling_features.reshape(C, HW)

  wg8 = jnp.zeros((8, C), jnp.float32).at[0].set(Wg[0])
  scores, down_hwc = _gate_scores(down_cm, wg8)
  top_idx = lax.top_k(scores[0], K_TOP)[1]
  idx_p = jnp.concatenate(
      [top_idx, jnp.broadcast_to(top_idx[K_TOP - 1:K_TOP], (K_PAD - K_TOP,))])

  d_sub = _sc_gather(down_hwc, idx_p, K_PAD // _SC_NW)

  sw_in = swinT_features.reshape(384, 56 * 56).T
  t0 = _conv1x1(sw_in, Wt.T, bt.reshape(1, C)).reshape(56, 56, C)
  t0p = jnp.pad(t0, ((1, 1), (1, 1), (0, 0)))
  w4_0 = Wct0.transpose(2, 3, 1, 0)
  u1ph = _convt0(t0p, w4_0, bct0.reshape(1, C))
  u1 = u1ph.reshape(2, 2, 56, 56, C).transpose(2, 0, 3, 1, 4).reshape(
      112, 112, C)
  u1p = jnp.pad(u1, ((1, 1), (1, 1), (0, 0)))
  t1 = _conv3x3_mid(u1p, Wc0.transpose(2, 3, 1, 0), bc0.reshape(1, C))
  t1p = jnp.pad(t1, ((1, 1), (1, 1), (0, 0)))

  w4_1 = jnp.pad(Wct1.transpose(2, 3, 1, 0), ((0, 0), (0, 0), (0, 0),
                                              (0, CP - C)))
  wp = jnp.stack([
      w4_1[py::2][:2][:, px::2][:, :2]
      for py in range(2) for px in range(2)
  ])
  u3ph = _convt1(t1p, wp, jnp.pad(bct1, (0, CP - C)).reshape(1, CP))
  u3flat = u3ph.reshape(HW, CP)

  y = idx_p // 224
  x = idx_p % 224
  d3 = jnp.arange(3, dtype=jnp.int32)
  yy = y[:, None, None] + d3[None, :, None] - 1
  xx = x[:, None, None] + d3[None, None, :] - 1
  valid = (yy >= 0) & (yy < 224) & (xx >= 0) & (xx < 224)
  yyc = jnp.clip(yy, 0, 223)
  xxc = jnp.clip(xx, 0, 223)
  ph = (yyc % 2) * 2 + (xxc % 2)
  gidx = (ph * (112 * 112) + (yyc // 2) * 112 + (xxc // 2)).reshape(-1)
  patches = _sc_gather(u3flat, gidx, 96).reshape(K_PAD, 9 * CP)
  mask = jnp.repeat(valid.reshape(K_PAD, 9).astype(jnp.float32), CP, axis=1)
  w3f = jnp.pad(Wc1.transpose(2, 3, 1, 0),
                ((0, 0), (0, 0), (0, CP - C), (0, 0))).reshape(9 * CP, C)
  s_sub = _patch_mm(patches, mask, w3f, bc1.reshape(1, C))

  wq = jnp.pad(in_proj_w[:C].T, ((0, CP - C), (0, 0)))
  wkv = in_proj_w[C:].T
  bq = in_proj_b[:C].reshape(1, C)
  bkv = in_proj_b[C:].reshape(1, 2 * C)
  att = _mha(d_sub, s_sub, wq, wkv, bq, bkv, out_w.T, out_b.reshape(1, C))

  att_p = jnp.pad(att, ((0, 0), (0, CP - C)))
  dest = jax.new_ref(down_hwc)
  _sc_scatter(dest, att_p, idx_p)
  return _untranspose(dest[...]).reshape(1, C, 224, 224)


# speedup vs baseline: 2.1874x; 1.0283x over previous
import functools
import math

import jax
import jax.numpy as jnp
from jax import lax
from jax.experimental import pallas as pl
from jax.experimental.pallas import tpu as pltpu
from jax.experimental.pallas import tpu_sc as plsc

NUM_HEADS = 8
K_TOP = 1000
K_PAD = 1024
C = 96
CP = 128
HW = 50176

_SC_NC = 2
_SC_NS = 16
_SC_NW = _SC_NC * _SC_NS


def _gate_body(x_ref, w_ref, s_ref, t_ref):
  x = x_ref[...]
  s_ref[...] = jnp.dot(w_ref[...], x, preferred_element_type=jnp.float32)
  t_ref[...] = jnp.concatenate(
      [x.T, jnp.zeros((x.shape[1], CP - C), jnp.float32)], axis=1)


def _gate_scores(down_cm, wg8):
  grid = 14
  blk = HW // grid
  return pl.pallas_call(
      _gate_body,
      grid=(grid,),
      in_specs=[
          pl.BlockSpec((C, blk), lambda j: (0, j)),
          pl.BlockSpec((8, C), lambda j: (0, 0)),
      ],
      out_specs=[
          pl.BlockSpec((8, blk), lambda j: (0, j)),
          pl.BlockSpec((blk, CP), lambda j: (j, 0)),
      ],
      out_shape=[
          jax.ShapeDtypeStruct((8, HW), jnp.float32),
          jax.ShapeDtypeStruct((HW, CP), jnp.float32),
      ],
  )(down_cm, wg8)


def _untranspose_body(x_ref, o_ref):
  o_ref[...] = x_ref[:, :C].T


def _untranspose(x):
  grid = 14
  blk = HW // grid
  return pl.pallas_call(
      _untranspose_body,
      grid=(grid,),
      in_specs=[pl.BlockSpec((blk, CP), lambda j: (j, 0))],
      out_specs=pl.BlockSpec((C, blk), lambda j: (0, j)),
      out_shape=jax.ShapeDtypeStruct((C, HW), jnp.float32),
  )(x)


def _mm_bias_body(x_ref, w_ref, b_ref, o_ref):
  acc = jnp.dot(x_ref[...].astype(jnp.bfloat16), w_ref[...],
                preferred_element_type=jnp.float32) + b_ref[...]
  o_ref[...] = acc.astype(jnp.bfloat16)


def _conv1x1(x, w, b):
  n, cin = x.shape
  cout = w.shape[1]
  return pl.pallas_call(
      _mm_bias_body,
      out_shape=jax.ShapeDtypeStruct((n, cout), jnp.bfloat16),
  )(x, w, b)


def _convt_small_body(x_ref, w_ref, b_ref, o_ref):
  for p in range(4):
    py, px = p // 2, p % 2
    acc = jnp.zeros((56 * 56, C), jnp.float32)
    for a in range(2):
      for bb in range(2):
        ry, rx = a - 1 + py, bb - 1 + px
        ky, kx = 2 * a + py, 2 * bb + px
        xs = x_ref[pl.ds(1 + ry, 56), pl.ds(1 + rx, 56), :]
        acc = acc + jnp.dot(xs.reshape(56 * 56, C), w_ref[ky, kx],
                            preferred_element_type=jnp.float32)
    o_ref[p] = jnp.maximum(acc + b_ref[...], 0.0).astype(
        jnp.bfloat16).reshape(56, 56, C)


def _convt0(xp, w4, b):
  return pl.pallas_call(
      _convt_small_body,
      out_shape=jax.ShapeDtypeStruct((4, 56, 56, C), jnp.bfloat16),
  )(xp, w4, b)


def _conv3x3_body(x_ref, w_ref, b_ref, o_ref):
  acc = jnp.zeros((112 * 112, C), jnp.float32)
  for dy in range(3):
    for dx in range(3):
      xs = x_ref[pl.ds(dy, 112), pl.ds(dx, 112), :]
      acc = acc + jnp.dot(xs.reshape(112 * 112, C), w_ref[dy, dx],
                          preferred_element_type=jnp.float32)
  o_ref[...] = (acc + b_ref[...]).reshape(112, 112, C)


def _conv3x3_mid(xp, w9, b):
  return pl.pallas_call(
      _conv3x3_body,
      out_shape=jax.ShapeDtypeStruct((112, 112, C), jnp.float32),
  )(xp, w9, b)


def _convt_big_body(x_ref, w_ref, b_ref, o_ref):
  p = pl.program_id(0)
  py, px = p // 2, p % 2
  acc = jnp.zeros((112 * 112, CP), jnp.float32)
  for a in range(2):
    for bb in range(2):
      xs = x_ref[pl.ds(a + py, 112), pl.ds(bb + px, 112), :]
      acc = acc + jnp.dot(xs.reshape(112 * 112, C).astype(jnp.bfloat16),
                          w_ref[0, a, bb],
                          preferred_element_type=jnp.float32)
  o_ref[0] = jnp.maximum(acc + b_ref[...], 0.0).reshape(112, 112, CP)


def _convt1(xp, wp, b):
  return pl.pallas_call(
      _convt_big_body,
      grid=(4,),
      in_specs=[
          pl.BlockSpec((114, 114, C), lambda p: (0, 0, 0)),
          pl.BlockSpec((1, 2, 2, C, CP), lambda p: (p, 0, 0, 0, 0)),
          pl.BlockSpec((1, CP), lambda p: (0, 0)),
      ],
      out_specs=pl.BlockSpec((1, 112, 112, CP), lambda p: (p, 0, 0, 0)),
      out_shape=jax.ShapeDtypeStruct((4, 112, 112, CP), jnp.float32),
  )(xp, wp, b)


def _patch_mm_body(x_ref, m_ref, w_ref, b_ref, o_ref):
  xm = (x_ref[...] * m_ref[...]).astype(jnp.bfloat16)
  o_ref[...] = jnp.dot(xm, w_ref[...],
                       preferred_element_type=jnp.float32) + b_ref[...]


def _patch_mm(patches, mask, w, b):
  return pl.pallas_call(
      _patch_mm_body,
      out_shape=jax.ShapeDtypeStruct((K_PAD, C), jnp.float32),
  )(patches, mask, w, b)


def _mha_body(d_ref, s_ref, wq_ref, wkv_ref, bq_ref, bkv_ref, wo_ref,
              bo_ref, o_ref):
  hd = C // NUM_HEADS
  scale = 1.0 / math.sqrt(hd)
  q = jnp.dot(d_ref[...].astype(jnp.bfloat16), wq_ref[...],
              preferred_element_type=jnp.float32) + bq_ref[...]
  kv = jnp.dot(s_ref[...].astype(jnp.bfloat16), wkv_ref[...],
               preferred_element_type=jnp.float32) + bkv_ref[...]
  col = lax.broadcasted_iota(jnp.int32, (K_PAD, K_PAD), 1)
  kv_mask = col < K_TOP
  outs = []
  for h in range(NUM_HEADS):
    qh = q[:, h * hd:(h + 1) * hd].astype(jnp.bfloat16)
    kh = kv[:, h * hd:(h + 1) * hd].astype(jnp.bfloat16)
    vh = kv[:, C + h * hd:C + (h + 1) * hd].astype(jnp.bfloat16)
    s = lax.dot_general(qh, kh, (((1,), (1,)), ((), ())),
                        preferred_element_type=jnp.float32) * scale
    s = jnp.where(kv_mask, s, -1e30)
    m = jnp.max(s, axis=-1, keepdims=True)
    e = jnp.exp(s - m)
    p = (e / jnp.sum(e, axis=-1, keepdims=True)).astype(jnp.bfloat16)
    outs.append(jnp.dot(p, vh, preferred_element_type=jnp.float32))
  o = jnp.concatenate(outs, axis=1).astype(jnp.bfloat16)
  o_ref[...] = jnp.dot(o, wo_ref[...],
                       preferred_element_type=jnp.float32) + bo_ref[...]


def _mha(d_sub, s_sub, wq, wkv, bq, bkv, wo, bo):
  return pl.pallas_call(
      _mha_body,
      out_shape=jax.ShapeDtypeStruct((K_PAD, C), jnp.float32),
  )(d_sub, s_sub, wq, wkv, bq, bkv, wo, bo)


def _sc_mesh():
  return plsc.VectorSubcoreMesh(core_axis_name="c", subcore_axis_name="s",
                                num_cores=_SC_NC, num_subcores=_SC_NS)


@functools.partial(jax.jit, static_argnums=(2,))
def _sc_gather(table, idx, chunk):
  b = idx.shape[0]
  per_w = b // _SC_NW
  chunks = per_w // chunk

  @functools.partial(
      pl.kernel,
      mesh=_sc_mesh(),
      out_type=jax.ShapeDtypeStruct((b, CP), jnp.float32),
      scratch_types=[
          pltpu.VMEM((chunk,), jnp.int32),
          pltpu.VMEM((chunk, CP), jnp.float32),
          pltpu.SemaphoreType.DMA,
      ],
  )
  def gk(table_hbm, idx_hbm, out_hbm, idx_v, rows_v, sem):
    wid = lax.axis_index("s") * _SC_NC + lax.axis_index("c")
    base = wid * per_w
    for ci in range(chunks):
      off = base + ci * chunk
      pltpu.sync_copy(idx_hbm.at[pl.ds(off, chunk)], idx_v)
      pltpu.async_copy(table_hbm.at[idx_v], rows_v, sem).wait()
      pltpu.sync_copy(rows_v, out_hbm.at[pl.ds(off, chunk)])

  return gk(table, idx)


def _sc_scatter(dest_ref, att, idx):
  per_w = K_PAD // _SC_NW

  @functools.partial(
      pl.kernel,
      mesh=_sc_mesh(),
      out_type=(),
      scratch_types=[
          pltpu.VMEM((per_w,), jnp.int32),
          pltpu.VMEM((per_w, CP), jnp.float32),
          pltpu.SemaphoreType.DMA,
      ],
  )
  def sk(att_hbm, idx_hbm, dest_hbm, idx_v, rows_v, sem):
    wid = lax.axis_index("s") * _SC_NC + lax.axis_index("c")
    base = wid * per_w
    pltpu.sync_copy(idx_hbm.at[pl.ds(base, per_w)], idx_v)
    pltpu.sync_copy(att_hbm.at[pl.ds(base, per_w)], rows_v)
    pltpu.async_copy(rows_v, dest_hbm.at[idx_v], sem).wait()

  sk(att, idx, dest_ref)


def kernel(downsampling_features, swinT_features, Wt, bt, Wct0, bct0, Wc0,
           bc0, Wct1, bct1, Wc1, bc1, Wg, bg, in_proj_w, in_proj_b, out_w,
           out_b):
  down_cm = downsampling_features.reshape(C, HW)

  wg8 = jnp.zeros((8, C), jnp.float32).at[0].set(Wg[0])
  scores, down_hwc = _gate_scores(down_cm, wg8)
  top_idx = lax.top_k(scores[0], K_TOP)[1]
  idx_p = jnp.concatenate(
      [top_idx, jnp.broadcast_to(top_idx[K_TOP - 1:K_TOP], (K_PAD - K_TOP,))])

  d_sub = _sc_gather(down_hwc, idx_p, K_PAD // _SC_NW)

  bf16 = jnp.bfloat16
  sw_in = swinT_features.reshape(384, 56 * 56).T
  t0 = _conv1x1(sw_in, Wt.T.astype(bf16), bt.reshape(1, C)).reshape(
      56, 56, C)
  t0p = jnp.pad(t0, ((1, 1), (1, 1), (0, 0)))
  w4_0 = Wct0.transpose(2, 3, 1, 0).astype(bf16)
  u1ph = _convt0(t0p, w4_0, bct0.reshape(1, C))
  u1 = u1ph.reshape(2, 2, 56, 56, C).transpose(2, 0, 3, 1, 4).reshape(
      112, 112, C)
  u1p = jnp.pad(u1, ((1, 1), (1, 1), (0, 0)))
  t1 = _conv3x3_mid(u1p, Wc0.transpose(2, 3, 1, 0).astype(bf16),
                    bc0.reshape(1, C))
  t1p = jnp.pad(t1, ((1, 1), (1, 1), (0, 0)))

  w4_1 = jnp.pad(Wct1.transpose(2, 3, 1, 0), ((0, 0), (0, 0), (0, 0),
                                              (0, CP - C))).astype(bf16)
  wp = jnp.stack([
      w4_1[py::2][:2][:, px::2][:, :2]
      for py in range(2) for px in range(2)
  ])
  u3ph = _convt1(t1p, wp, jnp.pad(bct1, (0, CP - C)).reshape(1, CP))
  u3flat = u3ph.reshape(HW, CP)

  y = idx_p // 224
  x = idx_p % 224
  d3 = jnp.arange(3, dtype=jnp.int32)
  yy = y[:, None, None] + d3[None, :, None] - 1
  xx = x[:, None, None] + d3[None, None, :] - 1
  valid = (yy >= 0) & (yy < 224) & (xx >= 0) & (xx < 224)
  yyc = jnp.clip(yy, 0, 223)
  xxc = jnp.clip(xx, 0, 223)
  ph = (yyc % 2) * 2 + (xxc % 2)
  gidx = (ph * (112 * 112) + (yyc // 2) * 112 + (xxc // 2)).reshape(-1)
  patches = _sc_gather(u3flat, gidx, 96).reshape(K_PAD, 9 * CP)
  mask = jnp.repeat(valid.reshape(K_PAD, 9).astype(jnp.float32), CP, axis=1)
  w3f = jnp.pad(Wc1.transpose(2, 3, 1, 0),
                ((0, 0), (0, 0), (0, CP - C),
                 (0, 0))).reshape(9 * CP, C).astype(bf16)
  s_sub = _patch_mm(patches, mask, w3f, bc1.reshape(1, C))

  wq = jnp.pad(in_proj_w[:C].T, ((0, CP - C), (0, 0))).astype(bf16)
  wkv = in_proj_w[C:].T.astype(bf16)
  bq = in_proj_b[:C].reshape(1, C)
  bkv = in_proj_b[C:].reshape(1, 2 * C)
  att = _mha(d_sub, s_sub, wq, wkv, bq, bkv, out_w.T.astype(bf16),
             out_b.reshape(1, C))

  att_p = jnp.pad(att, ((0, 0), (0, CP - C)))
  dest = jax.new_ref(down_hwc)
  _sc_scatter(dest, att_p, idx_p)
  return _untranspose(dest[...]).reshape(1, C, 224, 224)
